# baseline (speedup 1.0000x reference)
import functools

import jax
import jax.numpy as jnp
from jax import lax
from jax.experimental import pallas as pl
from jax.experimental.pallas import tpu as pltpu
from jax.experimental.pallas import tpu_sc as plsc

B = 16384
D = 64
NFEAT = 5
OUT_W = 4 * D + NFEAT

NC = 2
NS = 16
L = 16
NW = NC * NS
BPW = B // NW
CH = 128
NCH = BPW // CH


def _fire_gathers(table, idx_v, t, buf, sem):
  return [
      pltpu.async_copy(
          table.at[idx_v.at[t, pl.ds(j * CH, CH)]],
          buf.at[pl.ds(j * CH, CH), :], sem)
      for j in range(NCH)
  ]


def _body(weapon, rank, sub_w, spec_w, level, wrange, wpower, wrounds,
          wiine, W_weapon, W_rank, W_sub, W_special, out,
          idx_v, rows_a, rows_b, feats_v, sbuf_v, isem, gsem, osem):
  wid = lax.axis_index("s") * NC + lax.axis_index("c")
  base = wid * BPW

  stage = []
  for i, ref in enumerate((weapon, rank, sub_w, spec_w)):
    stage.append(pltpu.async_copy(ref.at[pl.ds(base, BPW)], idx_v.at[i],
                                  isem))
  for f, ref in enumerate((level, wrange, wpower, wrounds, wiine)):
    stage.append(pltpu.async_copy(ref.at[pl.ds(base, BPW)], feats_v.at[f],
                                  isem))
  for c in stage:
    c.wait()

  tables = (W_weapon, W_rank, W_sub, W_special)
  bufs = (rows_a, rows_b)

  gathers = _fire_gathers(tables[0], idx_v, 0, bufs[0], gsem)
  out_dmas = [None, None]
  for t in range(4):
    nxt = None
    if t + 1 < 4:
      if out_dmas[(t + 1) % 2] is not None:
        out_dmas[(t + 1) % 2].wait()
        out_dmas[(t + 1) % 2] = None
      nxt = _fire_gathers(tables[t + 1], idx_v, t + 1, bufs[(t + 1) % 2],
                          gsem)
    if t == 0:
      for f in range(NFEAT):
        col = jnp.full((L,), f, jnp.int32)
        for j in range(BPW // L):
          vals = feats_v[f, pl.ds(j * L, L)]
          rows = lax.iota(jnp.int32, L) + (j * L)
          plsc.store_scatter(sbuf_v, [rows, col], vals)
    for c in gathers:
      c.wait()
    out_dmas[t % 2] = pltpu.async_copy(
        bufs[t % 2], out.at[pl.ds(base, BPW), pl.ds(t * D, D)], osem)
    gathers = nxt

  pltpu.sync_copy(sbuf_v, out.at[pl.ds(base, BPW), pl.ds(4 * D, NFEAT)])
  for d in out_dmas:
    if d is not None:
      d.wait()


_embed = functools.partial(
    pl.kernel,
    out_type=jax.ShapeDtypeStruct((B, OUT_W), jnp.float32),
    mesh=plsc.VectorSubcoreMesh(core_axis_name="c", subcore_axis_name="s"),
    compiler_params=pltpu.CompilerParams(use_tc_tiling_on_sc=False,
                                         needs_layout_passes=False),
    scratch_types=[
        pltpu.VMEM((4, BPW), jnp.int32),
        pltpu.VMEM((BPW, D), jnp.float32),
        pltpu.VMEM((BPW, D), jnp.float32),
        pltpu.VMEM((NFEAT, BPW), jnp.float32),
        pltpu.VMEM((BPW, NFEAT), jnp.float32),
        pltpu.SemaphoreType.DMA,
        pltpu.SemaphoreType.DMA,
        pltpu.SemaphoreType.DMA,
    ],
)(_body)


V = 100000
TJ_FULL = V // 128
V_TAIL = V - TJ_FULL * 128


NB = 4
SW = NB * 128


def _t_block(stage_b, pbuf_v, ncb, boff, lane, fvs):
  def cblk(cb, _):
    base = cb * 16
    for k in range(16):
      sv = base + ((lane + k) & 15)
      rv = lax.shift_right_logical(sv, 1) + boff
      cpar = lax.shift_left(sv & 1, 6)
      for rb in range(4):
        g = plsc.load_gather(stage_b, [fvs[rb], sv])
        plsc.store_scatter(pbuf_v, [rv, cpar + fvs[rb]], g)
    return 0
  lax.fori_loop(0, ncb, cblk, 0)


def _t_body(wt, p, stage_v, tail_v, pbuf_v, ssem):
  wid = lax.axis_index("s") * NC + lax.axis_index("c")
  cnt = jnp.where(wid < 14, 25, 24) - jnp.where(wid == 31, 1, 0)
  j0 = wid * 24 + jnp.minimum(wid, 14)
  lane = lax.iota(jnp.int32, 16)
  fvs = [lane + 16 * rb for rb in range(4)]

  nblk = lax.div(cnt + NB - 1, NB)

  def loop_body(i, _):
    j = j0 + jnp.minimum(i * NB, cnt - NB)
    stages = [
        pltpu.async_copy(
            wt.at[:, pl.ds(pl.multiple_of((j + b) * 128, 128), 128)],
            stage_v.at[b], ssem)
        for b in range(NB)
    ]
    for c in stages:
      c.wait()
    for b in range(NB):
      _t_block(stage_v.at[b], pbuf_v, 8, b * 64, lane, fvs)
    pltpu.sync_copy(pbuf_v,
                    p.at[pl.ds(pl.multiple_of(j * 64, 64), NB * 64), :])
    return 0
  lax.fori_loop(0, nblk, loop_body, 0)

  @pl.when(wid == 31)
  def _tail():
    pltpu.sync_copy(wt.at[:, pl.ds(TJ_FULL * 128, V_TAIL)], tail_v)
    _t_block(tail_v, pbuf_v, V_TAIL // 16, 0, lane, fvs)
    pltpu.sync_copy(pbuf_v.at[pl.ds(0, V_TAIL // 2), :],
                    p.at[pl.ds(TJ_FULL * 64, V_TAIL // 2), :])


_transposer = functools.partial(
    pl.kernel,
    out_type=jax.ShapeDtypeStruct((V // 2, 128), jnp.float32),
    mesh=plsc.VectorSubcoreMesh(core_axis_name="c", subcore_axis_name="s"),
    compiler_params=pltpu.CompilerParams(use_tc_tiling_on_sc=True,
                                         needs_layout_passes=False),
    scratch_types=[
        pltpu.VMEM((NB, 64, 128), jnp.float32),
        pltpu.VMEM((64, V_TAIL), jnp.float32),
        pltpu.VMEM((NB * 64, 128), jnp.float32),
        pltpu.SemaphoreType.DMA,
    ],
)(_t_body)


def kernel(weapon, rank, level, sub_weapon, special_weapon, weapon_range,
           weapon_power, weapon_rounds_per, weapon_iine,
           W_weapon, W_rank, W_sub, W_special):
  p = _transposer(jnp.transpose(W_weapon))
  w_lin = jnp.reshape(p, (V, D))
  return _embed(weapon, rank, sub_weapon, special_weapon, level,
                weapon_range, weapon_power, weapon_rounds_per, weapon_iine,
                w_lin, W_rank, W_sub, W_special)


# speedup vs baseline: 1.5893x; 1.5893x over previous
import functools

import jax
import jax.numpy as jnp
from jax import lax
from jax.experimental import pallas as pl
from jax.experimental.pallas import tpu as pltpu
from jax.experimental.pallas import tpu_sc as plsc

B = 16384
D = 64
NFEAT = 5
OUT_W = 4 * D + NFEAT

NC = 2
NS = 16
L = 16
NW = NC * NS
BPW = B // NW
CH = 128
NCH = BPW // CH


def _fire_gathers(table, idx_v, t, buf, sem):
  return [
      pltpu.async_copy(
          table.at[idx_v.at[t, pl.ds(j * CH, CH)]],
          buf.at[pl.ds(j * CH, CH), :], sem)
      for j in range(NCH)
  ]


def _body(weapon, rank, sub_w, spec_w, level, wrange, wpower, wrounds,
          wiine, W_weapon, W_rank, W_sub, W_special, out,
          idx_v, rows_a, rows_b, feats_v, sbuf_v, isem, gsem, osem):
  wid = lax.axis_index("s") * NC + lax.axis_index("c")
  base = wid * BPW

  stage = []
  for i, ref in enumerate((weapon, rank, sub_w, spec_w)):
    stage.append(pltpu.async_copy(ref.at[pl.ds(base, BPW)], idx_v.at[i],
                                  isem))
  for f, ref in enumerate((level, wrange, wpower, wrounds, wiine)):
    stage.append(pltpu.async_copy(ref.at[pl.ds(base, BPW)], feats_v.at[f],
                                  isem))
  for c in stage:
    c.wait()

  tables = (W_weapon, W_rank, W_sub, W_special)
  bufs = (rows_a, rows_b)

  gathers = _fire_gathers(tables[0], idx_v, 0, bufs[0], gsem)
  out_dmas = [None, None]
  for t in range(4):
    nxt = None
    if t + 1 < 4:
      if out_dmas[(t + 1) % 2] is not None:
        out_dmas[(t + 1) % 2].wait()
        out_dmas[(t + 1) % 2] = None
      nxt = _fire_gathers(tables[t + 1], idx_v, t + 1, bufs[(t + 1) % 2],
                          gsem)
    if t == 0:
      for f in range(NFEAT):
        col = jnp.full((L,), f, jnp.int32)
        for j in range(BPW // L):
          vals = feats_v[f, pl.ds(j * L, L)]
          rows = lax.iota(jnp.int32, L) + (j * L)
          plsc.store_scatter(sbuf_v, [rows, col], vals)
    for c in gathers:
      c.wait()
    out_dmas[t % 2] = pltpu.async_copy(
        bufs[t % 2], out.at[pl.ds(base, BPW), pl.ds(t * D, D)], osem)
    gathers = nxt

  pltpu.sync_copy(sbuf_v, out.at[pl.ds(base, BPW), pl.ds(4 * D, NFEAT)])
  for d in out_dmas:
    if d is not None:
      d.wait()


_embed = functools.partial(
    pl.kernel,
    out_type=jax.ShapeDtypeStruct((B, OUT_W), jnp.float32),
    mesh=plsc.VectorSubcoreMesh(core_axis_name="c", subcore_axis_name="s"),
    compiler_params=pltpu.CompilerParams(use_tc_tiling_on_sc=False,
                                         needs_layout_passes=False),
    scratch_types=[
        pltpu.VMEM((4, BPW), jnp.int32),
        pltpu.VMEM((BPW, D), jnp.float32),
        pltpu.VMEM((BPW, D), jnp.float32),
        pltpu.VMEM((NFEAT, BPW), jnp.float32),
        pltpu.VMEM((BPW, NFEAT), jnp.float32),
        pltpu.SemaphoreType.DMA,
        pltpu.SemaphoreType.DMA,
        pltpu.SemaphoreType.DMA,
    ],
)(_body)


V = 100000
TJ_FULL = V // 128
V_TAIL = V - TJ_FULL * 128


NB = 4
SW = NB * 128


def _t_block(stage_b, pbuf_v, ncb, boff, lane, fvs):
  @plsc.parallel_loop(0, ncb * 16, 1, unroll=4)
  def diag(i):
    k = i & 15
    sv = (i - k) + ((lane + k) & 15)
    rv = lax.shift_right_logical(sv, 1) + boff
    cpar = lax.shift_left(sv & 1, 6)
    for rb in range(4):
      g = plsc.load_gather(stage_b, [fvs[rb], sv])
      plsc.store_scatter(pbuf_v, [rv, cpar + fvs[rb]], g)


def _t_body(wt, p, stage_v, tail_v, pbuf_v, ssem):
  wid = lax.axis_index("s") * NC + lax.axis_index("c")
  cnt = jnp.where(wid < 14, 25, 24) - jnp.where(wid == 31, 1, 0)
  j0 = wid * 24 + jnp.minimum(wid, 14)
  lane = lax.iota(jnp.int32, 16)
  fvs = [lane + 16 * rb for rb in range(4)]

  nblk = lax.div(cnt + NB - 1, NB)

  def loop_body(i, _):
    j = j0 + jnp.minimum(i * NB, cnt - NB)
    stages = [
        pltpu.async_copy(
            wt.at[:, pl.ds(pl.multiple_of((j + b) * 128, 128), 128)],
            stage_v.at[b], ssem)
        for b in range(NB)
    ]
    for c in stages:
      c.wait()
    for b in range(NB):
      _t_block(stage_v.at[b], pbuf_v, 8, b * 64, lane, fvs)
    pltpu.sync_copy(pbuf_v,
                    p.at[pl.ds(pl.multiple_of(j * 64, 64), NB * 64), :])
    return 0
  lax.fori_loop(0, nblk, loop_body, 0)

  @pl.when(wid == 31)
  def _tail():
    pltpu.sync_copy(wt.at[:, pl.ds(TJ_FULL * 128, V_TAIL)], tail_v)
    _t_block(tail_v, pbuf_v, V_TAIL // 16, 0, lane, fvs)
    pltpu.sync_copy(pbuf_v.at[pl.ds(0, V_TAIL // 2), :],
                    p.at[pl.ds(TJ_FULL * 64, V_TAIL // 2), :])


_transposer = functools.partial(
    pl.kernel,
    out_type=jax.ShapeDtypeStruct((V // 2, 128), jnp.float32),
    mesh=plsc.VectorSubcoreMesh(core_axis_name="c", subcore_axis_name="s"),
    compiler_params=pltpu.CompilerParams(use_tc_tiling_on_sc=True,
                                         needs_layout_passes=False),
    scratch_types=[
        pltpu.VMEM((NB, 64, 128), jnp.float32),
        pltpu.VMEM((64, V_TAIL), jnp.float32),
        pltpu.VMEM((NB * 64, 128), jnp.float32),
        pltpu.SemaphoreType.DMA,
    ],
)(_t_body)


def kernel(weapon, rank, level, sub_weapon, special_weapon, weapon_range,
           weapon_power, weapon_rounds_per, weapon_iine,
           W_weapon, W_rank, W_sub, W_special):
  p = _transposer(jnp.transpose(W_weapon))
  w_lin = jnp.reshape(p, (V, D))
  return _embed(weapon, rank, sub_weapon, special_weapon, level,
                weapon_range, weapon_power, weapon_rounds_per, weapon_iine,
                w_lin, W_rank, W_sub, W_special)
